# baseline (speedup 1.0000x reference)
import functools

import jax
import jax.numpy as jnp
from jax import lax
from jax.experimental import pallas as pl
from jax.experimental.pallas import tpu as pltpu
from jax.experimental.pallas import tpu_sc as plsc

_N = 16384
_D = 128
_NC = 2
_NS = 16
_LANES = 16
_RADIX = 256


_P0 = (-5.99633501014107895267E1, 9.80010754185999661536E1,
       -5.66762857469070293439E1, 1.39312609387279679503E1,
       -1.23916583867381258016E0)
_Q0 = (1.0, 1.95448858338141759834E0, 4.67627912898881538453E0,
       8.63602421390890590575E1, -2.25462687854119370527E2,
       2.00260212380060660359E2, -8.20372256168333339912E1,
       1.59056225126211695515E1, -1.18331621121330003142E0)
_P1 = (4.05544892305962419923E0, 3.15251094599893866154E1,
       5.71628192246421288162E1, 4.40805073893200834700E1,
       1.46849561928858024014E1, 2.18663306850790267539E0,
       -1.40256079171354495875E-1, -3.50424626827848203418E-2,
       -8.57456785154685413611E-4)
_Q1 = (1.0, 1.57799883256466749731E1, 4.53907635128879210584E1,
       4.13172038254672030440E1, 1.50425385692907503408E1,
       2.50464946208309415979E0, -1.42182922854787788574E-1,
       -3.80806407691578277194E-2, -9.33259480895457427372E-4)
_P2 = (3.23774891776946035970E0, 6.91522889068984211695E0,
       3.93881025292474443415E0, 1.33303460815807542389E0,
       2.01485389549179081538E-1, 1.23716634817820021358E-2,
       3.01581553508235416007E-4, 2.65806974686737550832E-6,
       6.23974539184983293730E-9)
_Q2 = (1.0, 6.02427039364742014255E0, 3.67983563856160859403E0,
       1.37702099489081330271E0, 2.16236993594496635890E-1,
       1.34204006088543189037E-2, 3.28014464682127739104E-4,
       2.89247864745380683936E-6, 6.79019408009981274425E-9)
_EXPM2 = 0.1353352832366127
_S2PI = 2.5066282746310002


def _polyval(coefs, x):
    r = jnp.full_like(x, coefs[0])
    for c in coefs[1:]:
        r = r * x + c
    return r


def _ndtri(p):
    mcp = jnp.where(p > 1.0 - _EXPM2, 1.0 - p, p)
    w = mcp - 0.5
    ww = w * w
    big = (w + w * ww * (_polyval(_P0, ww) / _polyval(_Q0, ww))) * (-_S2PI)
    z = jnp.sqrt(-2.0 * jnp.log(mcp))
    ft = z - jnp.log(z) / z
    zi = 1.0 / z
    small = ft - _polyval(_P2, zi) / _polyval(_Q2, zi) * zi
    other = ft - _polyval(_P1, zi) / _polyval(_Q1, zi) * zi
    x = jnp.where(mcp > _EXPM2, big, jnp.where(z >= 8.0, small, other))
    return jnp.where(p > 1.0 - _EXPM2, x, -x)


def _icdf_table(n):
    half = n // 2
    rows = half // 128

    def body(q_ref):
        r = lax.broadcasted_iota(jnp.int32, (rows, 128), 0)
        c = lax.broadcasted_iota(jnp.int32, (rows, 128), 1)
        k = (r * 128 + c + half).astype(jnp.float32)
        u = (k + 1.0) / float(n + 1)
        q_ref[...] = _ndtri(u)

    q = pl.pallas_call(
        body, out_shape=jax.ShapeDtypeStruct((rows, 128), jnp.float32))()
    return q.reshape(half)


def _transpose_in(x, n, d, blk=512):
    def body(x_ref, o_ref):
        o_ref[...] = lax.bitcast_convert_type(x_ref[...], jnp.int32).T

    return pl.pallas_call(
        body,
        grid=(n // blk,),
        in_specs=[pl.BlockSpec((blk, d), lambda i: (i, 0))],
        out_specs=pl.BlockSpec((d, blk), lambda i: (0, i)),
        out_shape=jax.ShapeDtypeStruct((d, n), jnp.int32),
    )(x)


def _transpose_out(zt, n, d, blk=512):
    def body(z_ref, o_ref):
        o_ref[...] = lax.bitcast_convert_type(z_ref[...].T, jnp.float32)

    return pl.pallas_call(
        body,
        grid=(n // blk,),
        in_specs=[pl.BlockSpec((d, blk), lambda i: (0, i))],
        out_specs=pl.BlockSpec((blk, d), lambda i: (i, 0)),
        out_shape=jax.ShapeDtypeStruct((n, d), jnp.float32),
    )(zt)


def _build_rank_kernel(n, d, nc=_NC, ns=_NS, interpret=False):
    nw = nc * ns
    cpw = d // nw
    pairs = cpw // 2
    vregs = n // _LANES
    shv = vregs.bit_length() - 1
    nsk = n + _LANES
    hh = _RADIX * _LANES
    pmask = n - 1
    psh = n.bit_length() - 1

    mesh = plsc.VectorSubcoreMesh(
        core_axis_name="c", subcore_axis_name="s",
        num_cores=nc, num_subcores=ns)

    @functools.partial(
        pl.kernel,
        out_type=jax.ShapeDtypeStruct((d, n), jnp.int32),
        mesh=mesh,
        interpret=interpret,
        compiler_params=pltpu.CompilerParams(needs_layout_passes=False),
        scratch_types=[
            pltpu.VMEM((nsk,), jnp.int32),
            pltpu.VMEM((nsk,), jnp.int32),
            pltpu.VMEM((nsk,), jnp.int32),
            pltpu.VMEM((nsk,), jnp.int32),
            pltpu.VMEM((nsk,), jnp.int32),
            pltpu.VMEM((nsk,), jnp.int32),
            pltpu.VMEM((hh + _LANES,), jnp.int32),
            pltpu.VMEM((hh + _LANES,), jnp.int32),
            pltpu.VMEM((hh + _LANES,), jnp.int32),
            pltpu.VMEM((hh + _LANES,), jnp.int32),
            pltpu.VMEM((n // 2,), jnp.float32),
        ],
    )
    def body(xt_hbm, q_hbm, out_hbm, a_s, a_k, a_p, b_s, b_k, b_p,
             h1a, h1b, h2a, h2b, qv):
        wid = lax.axis_index("s") * nc + lax.axis_index("c")
        lane = lax.iota(jnp.int32, _LANES)
        lane_s = lane * vregs
        lane_sp = lane * (vregs + 1)
        ones = jnp.ones((_LANES,), jnp.int32)

        pltpu.sync_copy(q_hbm, qv)

        def skew(pos):
            return pos + (pos >> shv)

        def to_key(b):
            return b ^ ((b >> 31) & jnp.int32(0x7FFFFFFF))

        dig = (
            lambda k: k & 255,
            lambda k: (k >> 8) & 255,
            lambda k: (k >> psh) & 255,
            lambda k: ((k >> (psh + 8)) & 255) ^ 128,
        )

        def relayout2(sa, sb):
            def pre(j):
                return (to_key(sa[pl.ds(j * _LANES, _LANES)]),
                        to_key(sb[pl.ds(j * _LANES, _LANES)]))

            def cnt(j, ka, kb):
                lp = (j * _LANES + lane) >> shv
                plsc.addupdate_scatter(h1a, [dig[0](ka) * _LANES + lp], ones)
                plsc.addupdate_scatter(h1b, [dig[0](kb) * _LANES + lp], ones)

            def rb(i, carry):
                ka, kb = carry
                j = vregs - 1 - i
                nxt = pre(j - 1)
                fo = skew(j * _LANES + lane)
                plsc.store_scatter(sa, [fo], ka)
                plsc.store_scatter(sb, [fo], kb)
                cnt(j, ka, kb)
                return nxt
            ka, kb = lax.fori_loop(0, vregs - 1, rb, pre(vregs - 1),
                                   unroll=2)
            plsc.store_scatter(sa, [lane], ka)
            plsc.store_scatter(sb, [lane], kb)
            cnt(0, ka, kb)

        def zero2(ha, hb):
            def zb(j, c):
                z16 = jnp.zeros((_LANES,), jnp.int32)
                ha[pl.ds(j * _LANES, _LANES)] = z16
                hb[pl.ds(j * _LANES, _LANES)] = z16
                return c
            lax.fori_loop(0, _RADIX, zb, 0, unroll=8)

        def gat(ref, j):
            return plsc.load_gather(ref, [lane_sp + j])

        def scan2(cur_a, cur_b, zro_a, zro_b):
            def sb(j, carry):
                ca, cb, ha, hb = carry
                sa = plsc.cumsum(ha)
                sb_ = plsc.cumsum(hb)
                nha = cur_a[pl.ds((j + 1) * _LANES, _LANES)]
                nhb = cur_b[pl.ds((j + 1) * _LANES, _LANES)]
                z16 = jnp.zeros((_LANES,), jnp.int32)
                zro_a[pl.ds(j * _LANES, _LANES)] = z16
                zro_b[pl.ds(j * _LANES, _LANES)] = z16
                cur_a[pl.ds(j * _LANES, _LANES)] = sa - ha + ca
                cur_b[pl.ds(j * _LANES, _LANES)] = sb_ - hb + cb
                return (ca + sa[_LANES - 1], cb + sb_[_LANES - 1],
                        nha, nhb)
            lax.fori_loop(0, _RADIX, sb,
                          (jnp.int32(0), jnp.int32(0),
                           cur_a[pl.ds(0, _LANES)],
                           cur_b[pl.ds(0, _LANES)]),
                          unroll=2)

        def perm2(cur_a, cur_b, nxt, ndg, pref_a, pref_b, emit_a, emit_b):
            def pb(j, carry):
                (ka, ha), (kb, hb) = carry
                ofs_a = plsc.load_gather(cur_a, [ha])
                na = pref_a(j + 1)
                ofs_b = plsc.load_gather(cur_b, [hb])
                nb = pref_b(j + 1)
                emit_a(j, ka, ofs_a)
                plsc.store_scatter(cur_a, [ha], ofs_a + 1)
                if nxt is not None:
                    plsc.addupdate_scatter(
                        nxt[0], [ndg(ka) * _LANES + (ofs_a >> shv)], ones)
                emit_b(j, kb, ofs_b)
                plsc.store_scatter(cur_b, [hb], ofs_b + 1)
                if nxt is not None:
                    plsc.addupdate_scatter(
                        nxt[1], [ndg(kb) * _LANES + (ofs_b >> shv)], ones)
                return (na, nb)
            lax.fori_loop(0, vregs, pb, (pref_a(0), pref_b(0)), unroll=2)

        def move_pref(src, dg):
            def pref(j):
                k = gat(src, j)
                return (k, dg(k) * _LANES + lane)
            return pref

        def move_emit(kdst, pay=None):
            def emit(j, k, ofs):
                fo = skew(ofs)
                plsc.store_scatter(kdst, [fo], k)
                if pay is not None:
                    plsc.store_scatter(pay, [fo], lane_s + j)
            return emit

        def pack_pref(ksrc, psrc):
            def pref(j):
                k = gat(ksrc, j)
                p = gat(psrc, j)
                pk = (((k >> 16) & 0xFFFF) << psh) | p
                return (pk, dig[1](k) * _LANES + lane)
            return pref

        def out_emit(outdst):
            half = n // 2

            def emit(j, pk, ofs):
                hi = ofs >= half
                m = jnp.where(hi, ofs - half, (half - 1) - ofs)
                z = plsc.load_gather(qv, [m])
                z = jnp.where(hi, z, -z)
                plsc.store_scatter(outdst, [pk & pmask],
                                   plsc.bitcast(z, jnp.int32))
            return emit

        for t in range(pairs):
            ca = wid * cpw + 2 * t
            cb = ca + 1
            pltpu.sync_copy(xt_hbm.at[ca], a_s.at[pl.ds(0, n)])
            pltpu.sync_copy(xt_hbm.at[cb], b_s.at[pl.ds(0, n)])
            if t == 0:
                zero2(h1a, h1b)
            relayout2(a_s, b_s)

            scan2(h1a, h1b, h2a, h2b)
            perm2(h1a, h1b, (h2a, h2b), dig[1],
                  move_pref(a_s, dig[0]), move_pref(b_s, dig[0]),
                  move_emit(a_k, pay=a_p), move_emit(b_k, pay=b_p))

            scan2(h2a, h2b, h1a, h1b)
            perm2(h2a, h2b, (h1a, h1b), dig[2],
                  pack_pref(a_k, a_p), pack_pref(b_k, b_p),
                  move_emit(a_s), move_emit(b_s))

            scan2(h1a, h1b, h2a, h2b)
            perm2(h1a, h1b, (h2a, h2b), dig[3],
                  move_pref(a_s, dig[2]), move_pref(b_s, dig[2]),
                  move_emit(a_k), move_emit(b_k))

            scan2(h2a, h2b, h1a, h1b)
            perm2(h2a, h2b, None, None,
                  move_pref(a_k, dig[3]), move_pref(b_k, dig[3]),
                  out_emit(a_p), out_emit(b_p))

            pltpu.sync_copy(a_p.at[pl.ds(0, n)], out_hbm.at[ca])
            pltpu.sync_copy(b_p.at[pl.ds(0, n)], out_hbm.at[cb])

    return body


def kernel(x):
    q = _icdf_table(_N)
    xt = _transpose_in(x, _N, _D)
    zt = _build_rank_kernel(_N, _D)(xt, q)
    return _transpose_out(zt, _N, _D)


# speedup vs baseline: 1.0209x; 1.0209x over previous
import functools

import jax
import jax.numpy as jnp
from jax import lax
from jax.experimental import pallas as pl
from jax.experimental.pallas import tpu as pltpu
from jax.experimental.pallas import tpu_sc as plsc

_N = 16384
_D = 128
_NC = 2
_NS = 16
_LANES = 16
_RADIX = 256


_P0 = (-5.99633501014107895267E1, 9.80010754185999661536E1,
       -5.66762857469070293439E1, 1.39312609387279679503E1,
       -1.23916583867381258016E0)
_Q0 = (1.0, 1.95448858338141759834E0, 4.67627912898881538453E0,
       8.63602421390890590575E1, -2.25462687854119370527E2,
       2.00260212380060660359E2, -8.20372256168333339912E1,
       1.59056225126211695515E1, -1.18331621121330003142E0)
_P1 = (4.05544892305962419923E0, 3.15251094599893866154E1,
       5.71628192246421288162E1, 4.40805073893200834700E1,
       1.46849561928858024014E1, 2.18663306850790267539E0,
       -1.40256079171354495875E-1, -3.50424626827848203418E-2,
       -8.57456785154685413611E-4)
_Q1 = (1.0, 1.57799883256466749731E1, 4.53907635128879210584E1,
       4.13172038254672030440E1, 1.50425385692907503408E1,
       2.50464946208309415979E0, -1.42182922854787788574E-1,
       -3.80806407691578277194E-2, -9.33259480895457427372E-4)
_P2 = (3.23774891776946035970E0, 6.91522889068984211695E0,
       3.93881025292474443415E0, 1.33303460815807542389E0,
       2.01485389549179081538E-1, 1.23716634817820021358E-2,
       3.01581553508235416007E-4, 2.65806974686737550832E-6,
       6.23974539184983293730E-9)
_Q2 = (1.0, 6.02427039364742014255E0, 3.67983563856160859403E0,
       1.37702099489081330271E0, 2.16236993594496635890E-1,
       1.34204006088543189037E-2, 3.28014464682127739104E-4,
       2.89247864745380683936E-6, 6.79019408009981274425E-9)
_EXPM2 = 0.1353352832366127
_S2PI = 2.5066282746310002


def _polyval(coefs, x):
    r = jnp.full_like(x, coefs[0])
    for c in coefs[1:]:
        r = r * x + c
    return r


def _ndtri(p):
    mcp = jnp.where(p > 1.0 - _EXPM2, 1.0 - p, p)
    w = mcp - 0.5
    ww = w * w
    big = (w + w * ww * (_polyval(_P0, ww) / _polyval(_Q0, ww))) * (-_S2PI)
    z = jnp.sqrt(-2.0 * jnp.log(mcp))
    ft = z - jnp.log(z) / z
    zi = 1.0 / z
    small = ft - _polyval(_P2, zi) / _polyval(_Q2, zi) * zi
    other = ft - _polyval(_P1, zi) / _polyval(_Q1, zi) * zi
    x = jnp.where(mcp > _EXPM2, big, jnp.where(z >= 8.0, small, other))
    return jnp.where(p > 1.0 - _EXPM2, x, -x)


def _icdf_table(n):
    half = n // 2
    rows = half // 128

    def body(q_ref):
        r = lax.broadcasted_iota(jnp.int32, (rows, 128), 0)
        c = lax.broadcasted_iota(jnp.int32, (rows, 128), 1)
        k = (r * 128 + c + half).astype(jnp.float32)
        u = (k + 1.0) / float(n + 1)
        q_ref[...] = _ndtri(u)

    q = pl.pallas_call(
        body, out_shape=jax.ShapeDtypeStruct((rows, 128), jnp.float32))()
    return q.reshape(half)


def _transpose_in(x, n, d, blk=512):
    def body(x_ref, o_ref):
        o_ref[...] = lax.bitcast_convert_type(x_ref[...], jnp.int32).T

    return pl.pallas_call(
        body,
        grid=(n // blk,),
        in_specs=[pl.BlockSpec((blk, d), lambda i: (i, 0))],
        out_specs=pl.BlockSpec((d, blk), lambda i: (0, i)),
        out_shape=jax.ShapeDtypeStruct((d, n), jnp.int32),
    )(x)


def _transpose_out(zt, n, d, blk=512):
    def body(z_ref, o_ref):
        o_ref[...] = lax.bitcast_convert_type(z_ref[...].T, jnp.float32)

    return pl.pallas_call(
        body,
        grid=(n // blk,),
        in_specs=[pl.BlockSpec((d, blk), lambda i: (0, i))],
        out_specs=pl.BlockSpec((blk, d), lambda i: (i, 0)),
        out_shape=jax.ShapeDtypeStruct((n, d), jnp.float32),
    )(zt)


def _build_rank_kernel(n, d, nc=_NC, ns=_NS, interpret=False):
    nw = nc * ns
    cpw = d // nw
    pairs = cpw // 2
    vregs = n // _LANES
    shv = vregs.bit_length() - 1
    nsk = n + _LANES
    hh = _RADIX * _LANES
    pmask = n - 1
    psh = n.bit_length() - 1

    mesh = plsc.VectorSubcoreMesh(
        core_axis_name="c", subcore_axis_name="s",
        num_cores=nc, num_subcores=ns)

    @functools.partial(
        pl.kernel,
        out_type=jax.ShapeDtypeStruct((d, n), jnp.int32),
        mesh=mesh,
        interpret=interpret,
        compiler_params=pltpu.CompilerParams(needs_layout_passes=False),
        scratch_types=[
            pltpu.VMEM((nsk,), jnp.int32),
            pltpu.VMEM((nsk,), jnp.int32),
            pltpu.VMEM((nsk,), jnp.int32),
            pltpu.VMEM((nsk,), jnp.int32),
            pltpu.VMEM((nsk,), jnp.int32),
            pltpu.VMEM((nsk,), jnp.int32),
            pltpu.VMEM((hh + _LANES,), jnp.int32),
            pltpu.VMEM((hh + _LANES,), jnp.int32),
            pltpu.VMEM((hh + _LANES,), jnp.int32),
            pltpu.VMEM((hh + _LANES,), jnp.int32),
            pltpu.VMEM((n // 2,), jnp.float32),
        ],
    )
    def body(xt_hbm, q_hbm, out_hbm, a_s, a_k, a_p, b_s, b_k, b_p,
             h1a, h1b, h2a, h2b, qv):
        wid = lax.axis_index("s") * nc + lax.axis_index("c")
        lane = lax.iota(jnp.int32, _LANES)
        lane_s = lane * vregs
        lane_sp = lane * (vregs + 1)
        ones = jnp.ones((_LANES,), jnp.int32)

        pltpu.sync_copy(q_hbm, qv)

        def skew(pos):
            return pos + (pos >> shv)

        def to_key(b):
            return b ^ ((b >> 31) & jnp.int32(0x7FFFFFFF))

        dig = (
            lambda k: k & 255,
            lambda k: (k >> 8) & 255,
            lambda k: (k >> psh) & 255,
            lambda k: ((k >> (psh + 8)) & 255) ^ 128,
        )

        def relayout2(sa, sb):
            def pre(j):
                return (to_key(sa[pl.ds(j * _LANES, _LANES)]),
                        to_key(sb[pl.ds(j * _LANES, _LANES)]))

            def cnt(j, ka, kb):
                lp = (j * _LANES + lane) >> shv
                plsc.addupdate_scatter(h1a, [dig[0](ka) * _LANES + lp], ones)
                plsc.addupdate_scatter(h1b, [dig[0](kb) * _LANES + lp], ones)

            def rb(i, carry):
                ka, kb = carry
                j = vregs - 1 - i
                nxt = pre(j - 1)
                fo = skew(j * _LANES + lane)
                plsc.store_scatter(sa, [fo], ka)
                plsc.store_scatter(sb, [fo], kb)
                cnt(j, ka, kb)
                return nxt
            ka, kb = lax.fori_loop(0, vregs - 1, rb, pre(vregs - 1),
                                   unroll=4)
            plsc.store_scatter(sa, [lane], ka)
            plsc.store_scatter(sb, [lane], kb)
            cnt(0, ka, kb)

        def zero2(ha, hb):
            def zb(j, c):
                z16 = jnp.zeros((_LANES,), jnp.int32)
                ha[pl.ds(j * _LANES, _LANES)] = z16
                hb[pl.ds(j * _LANES, _LANES)] = z16
                return c
            lax.fori_loop(0, _RADIX, zb, 0, unroll=8)

        def gat(ref, j):
            return plsc.load_gather(ref, [lane_sp + j])

        def scan2(cur_a, cur_b, zro_a, zro_b):
            def sb(j, carry):
                ca, cb, ha, hb = carry
                sa = plsc.cumsum(ha)
                sb_ = plsc.cumsum(hb)
                nha = cur_a[pl.ds((j + 1) * _LANES, _LANES)]
                nhb = cur_b[pl.ds((j + 1) * _LANES, _LANES)]
                z16 = jnp.zeros((_LANES,), jnp.int32)
                zro_a[pl.ds(j * _LANES, _LANES)] = z16
                zro_b[pl.ds(j * _LANES, _LANES)] = z16
                cur_a[pl.ds(j * _LANES, _LANES)] = sa - ha + ca
                cur_b[pl.ds(j * _LANES, _LANES)] = sb_ - hb + cb
                return (ca + sa[_LANES - 1], cb + sb_[_LANES - 1],
                        nha, nhb)
            lax.fori_loop(0, _RADIX, sb,
                          (jnp.int32(0), jnp.int32(0),
                           cur_a[pl.ds(0, _LANES)],
                           cur_b[pl.ds(0, _LANES)]),
                          unroll=2)

        def perm2(cur_a, cur_b, nxt, ndg, pref_a, pref_b, emit_a, emit_b):
            def pb(j, carry):
                (ka, ha), (kb, hb) = carry
                ofs_a = plsc.load_gather(cur_a, [ha])
                na = pref_a(j + 1)
                ofs_b = plsc.load_gather(cur_b, [hb])
                nb = pref_b(j + 1)
                emit_a(j, ka, ofs_a)
                plsc.store_scatter(cur_a, [ha], ofs_a + 1)
                if nxt is not None:
                    plsc.addupdate_scatter(
                        nxt[0], [ndg(ka) * _LANES + (ofs_a >> shv)], ones)
                emit_b(j, kb, ofs_b)
                plsc.store_scatter(cur_b, [hb], ofs_b + 1)
                if nxt is not None:
                    plsc.addupdate_scatter(
                        nxt[1], [ndg(kb) * _LANES + (ofs_b >> shv)], ones)
                return (na, nb)
            lax.fori_loop(0, vregs, pb, (pref_a(0), pref_b(0)), unroll=4)

        def move_pref(src, dg):
            def pref(j):
                k = gat(src, j)
                return (k, dg(k) * _LANES + lane)
            return pref

        def move_emit(kdst, pay=None):
            def emit(j, k, ofs):
                fo = skew(ofs)
                plsc.store_scatter(kdst, [fo], k)
                if pay is not None:
                    plsc.store_scatter(pay, [fo], lane_s + j)
            return emit

        def pack_pref(ksrc, psrc):
            def pref(j):
                k = gat(ksrc, j)
                p = gat(psrc, j)
                pk = (((k >> 16) & 0xFFFF) << psh) | p
                return (pk, dig[1](k) * _LANES + lane)
            return pref

        def out_emit(outdst):
            half = n // 2

            def emit(j, pk, ofs):
                hi = ofs >= half
                m = jnp.where(hi, ofs - half, (half - 1) - ofs)
                z = plsc.load_gather(qv, [m])
                z = jnp.where(hi, z, -z)
                plsc.store_scatter(outdst, [pk & pmask],
                                   plsc.bitcast(z, jnp.int32))
            return emit

        for t in range(pairs):
            ca = wid * cpw + 2 * t
            cb = ca + 1
            pltpu.sync_copy(xt_hbm.at[ca], a_s.at[pl.ds(0, n)])
            pltpu.sync_copy(xt_hbm.at[cb], b_s.at[pl.ds(0, n)])
            if t == 0:
                zero2(h1a, h1b)
            relayout2(a_s, b_s)

            scan2(h1a, h1b, h2a, h2b)
            perm2(h1a, h1b, (h2a, h2b), dig[1],
                  move_pref(a_s, dig[0]), move_pref(b_s, dig[0]),
                  move_emit(a_k, pay=a_p), move_emit(b_k, pay=b_p))

            scan2(h2a, h2b, h1a, h1b)
            perm2(h2a, h2b, (h1a, h1b), dig[2],
                  pack_pref(a_k, a_p), pack_pref(b_k, b_p),
                  move_emit(a_s), move_emit(b_s))

            scan2(h1a, h1b, h2a, h2b)
            perm2(h1a, h1b, (h2a, h2b), dig[3],
                  move_pref(a_s, dig[2]), move_pref(b_s, dig[2]),
                  move_emit(a_k), move_emit(b_k))

            scan2(h2a, h2b, h1a, h1b)
            perm2(h2a, h2b, None, None,
                  move_pref(a_k, dig[3]), move_pref(b_k, dig[3]),
                  out_emit(a_p), out_emit(b_p))

            pltpu.sync_copy(a_p.at[pl.ds(0, n)], out_hbm.at[ca])
            pltpu.sync_copy(b_p.at[pl.ds(0, n)], out_hbm.at[cb])

    return body


def kernel(x):
    q = _icdf_table(_N)
    xt = lax.bitcast_convert_type(x, jnp.int32).T
    zt = _build_rank_kernel(_N, _D)(xt, q)
    return lax.bitcast_convert_type(zt.T, jnp.float32)


# speedup vs baseline: 1.0739x; 1.0519x over previous
import functools

import jax
import jax.numpy as jnp
from jax import lax
from jax.experimental import pallas as pl
from jax.experimental.pallas import tpu as pltpu
from jax.experimental.pallas import tpu_sc as plsc

_N = 16384
_D = 128
_NC = 2
_NS = 16
_LANES = 16
_RADIX = 256


_P0 = (-5.99633501014107895267E1, 9.80010754185999661536E1,
       -5.66762857469070293439E1, 1.39312609387279679503E1,
       -1.23916583867381258016E0)
_Q0 = (1.0, 1.95448858338141759834E0, 4.67627912898881538453E0,
       8.63602421390890590575E1, -2.25462687854119370527E2,
       2.00260212380060660359E2, -8.20372256168333339912E1,
       1.59056225126211695515E1, -1.18331621121330003142E0)
_P1 = (4.05544892305962419923E0, 3.15251094599893866154E1,
       5.71628192246421288162E1, 4.40805073893200834700E1,
       1.46849561928858024014E1, 2.18663306850790267539E0,
       -1.40256079171354495875E-1, -3.50424626827848203418E-2,
       -8.57456785154685413611E-4)
_Q1 = (1.0, 1.57799883256466749731E1, 4.53907635128879210584E1,
       4.13172038254672030440E1, 1.50425385692907503408E1,
       2.50464946208309415979E0, -1.42182922854787788574E-1,
       -3.80806407691578277194E-2, -9.33259480895457427372E-4)
_P2 = (3.23774891776946035970E0, 6.91522889068984211695E0,
       3.93881025292474443415E0, 1.33303460815807542389E0,
       2.01485389549179081538E-1, 1.23716634817820021358E-2,
       3.01581553508235416007E-4, 2.65806974686737550832E-6,
       6.23974539184983293730E-9)
_Q2 = (1.0, 6.02427039364742014255E0, 3.67983563856160859403E0,
       1.37702099489081330271E0, 2.16236993594496635890E-1,
       1.34204006088543189037E-2, 3.28014464682127739104E-4,
       2.89247864745380683936E-6, 6.79019408009981274425E-9)
_EXPM2 = 0.1353352832366127
_S2PI = 2.5066282746310002


def _polyval(coefs, x):
    r = jnp.full_like(x, coefs[0])
    for c in coefs[1:]:
        r = r * x + c
    return r


def _ndtri(p):
    mcp = jnp.where(p > 1.0 - _EXPM2, 1.0 - p, p)
    w = mcp - 0.5
    ww = w * w
    big = (w + w * ww * (_polyval(_P0, ww) / _polyval(_Q0, ww))) * (-_S2PI)
    z = jnp.sqrt(-2.0 * jnp.log(mcp))
    ft = z - jnp.log(z) / z
    zi = 1.0 / z
    small = ft - _polyval(_P2, zi) / _polyval(_Q2, zi) * zi
    other = ft - _polyval(_P1, zi) / _polyval(_Q1, zi) * zi
    x = jnp.where(mcp > _EXPM2, big, jnp.where(z >= 8.0, small, other))
    return jnp.where(p > 1.0 - _EXPM2, x, -x)


def _icdf_table(n):
    half = n // 2
    rows = half // 128

    def body(q_ref):
        r = lax.broadcasted_iota(jnp.int32, (rows, 128), 0)
        c = lax.broadcasted_iota(jnp.int32, (rows, 128), 1)
        k = (r * 128 + c + half).astype(jnp.float32)
        u = (k + 1.0) / float(n + 1)
        q_ref[...] = _ndtri(u)

    q = pl.pallas_call(
        body, out_shape=jax.ShapeDtypeStruct((rows, 128), jnp.float32))()
    return q.reshape(half)


def _build_rank_kernel(n, d, nc=_NC, ns=_NS, interpret=False):
    nw = nc * ns
    cpw = d // nw
    pairs = cpw // 2
    vregs = n // _LANES
    vregs2 = vregs // 2
    shv2 = vregs2.bit_length() - 1
    nsk = n + 2 * _LANES
    hh2 = 2 * _RADIX * _LANES
    pmask = n - 1
    psh = n.bit_length() - 1
    half_n = n // 2

    mesh = plsc.VectorSubcoreMesh(
        core_axis_name="c", subcore_axis_name="s",
        num_cores=nc, num_subcores=ns)

    @functools.partial(
        pl.kernel,
        out_type=jax.ShapeDtypeStruct((d, n), jnp.int32),
        mesh=mesh,
        interpret=interpret,
        compiler_params=pltpu.CompilerParams(needs_layout_passes=False),
        scratch_types=[
            pltpu.VMEM((nsk,), jnp.int32),
            pltpu.VMEM((nsk,), jnp.int32),
            pltpu.VMEM((nsk,), jnp.int32),
            pltpu.VMEM((nsk,), jnp.int32),
            pltpu.VMEM((nsk,), jnp.int32),
            pltpu.VMEM((nsk,), jnp.int32),
            pltpu.VMEM((hh2 + _LANES,), jnp.int32),
            pltpu.VMEM((hh2 + _LANES,), jnp.int32),
            pltpu.VMEM((half_n,), jnp.float32),
        ],
    )
    def body(xt_hbm, q_hbm, out_hbm, a_s, a_k, a_p, b_s, b_k, b_p,
             h_a, h_b, qv):
        wid = lax.axis_index("s") * nc + lax.axis_index("c")
        lane = lax.iota(jnp.int32, _LANES)
        lane_sp = lane * (vregs2 + 1)
        ch_off = _LANES * (vregs2 + 1)
        lane_s2 = lane * vregs2
        ones = jnp.ones((_LANES,), jnp.int32)

        pltpu.sync_copy(q_hbm, qv)

        def skew(pos):
            return pos + (pos >> shv2)

        def to_key(b):
            return b ^ ((b >> 31) & jnp.int32(0x7FFFFFFF))

        dig = (
            lambda k: k & 255,
            lambda k: (k >> 8) & 255,
            lambda k: (k >> psh) & 255,
            lambda k: ((k >> (psh + 8)) & 255) ^ 128,
        )

        def pos_of(h, j):
            return lane_s2 + h * half_n + j

        def adr_of(h, j):
            return lane_sp + h * ch_off + j

        def relayout2(sa, sb):
            def pre(j):
                return (to_key(sa[pl.ds(j * _LANES, _LANES)]),
                        to_key(sb[pl.ds(j * _LANES, _LANES)]))

            def rb(i, carry):
                ka, kb = carry
                j = vregs - 1 - i
                nxt = pre(j - 1)
                fo = skew(j * _LANES + lane)
                plsc.store_scatter(sa, [fo], ka)
                plsc.store_scatter(sb, [fo], kb)
                return nxt
            ka, kb = lax.fori_loop(0, vregs - 1, rb, pre(vregs - 1),
                                   unroll=2)
            plsc.store_scatter(sa, [lane], ka)
            plsc.store_scatter(sb, [lane], kb)

        def zero2():
            def zb(j, c):
                z16 = jnp.zeros((_LANES,), jnp.int32)
                h_a[pl.ds(j * _LANES, _LANES)] = z16
                h_b[pl.ds(j * _LANES, _LANES)] = z16
                return c
            lax.fori_loop(0, 2 * _RADIX, zb, 0, unroll=8)

        def count2(sa, sb, dg):
            def hx(k, h):
                return dg(k) * (2 * _LANES) + lane + h * _LANES

            def cb(j, carry):
                ha0, ha1, hb0, hb1 = carry
                plsc.addupdate_scatter(h_a, [ha0], ones)
                na0 = plsc.load_gather(sa, [adr_of(0, j + 1)])
                plsc.addupdate_scatter(h_a, [ha1], ones)
                na1 = plsc.load_gather(sa, [adr_of(1, j + 1)])
                plsc.addupdate_scatter(h_b, [hb0], ones)
                nb0 = plsc.load_gather(sb, [adr_of(0, j + 1)])
                plsc.addupdate_scatter(h_b, [hb1], ones)
                nb1 = plsc.load_gather(sb, [adr_of(1, j + 1)])
                return (hx(na0, 0), hx(na1, 1), hx(nb0, 0), hx(nb1, 1))
            init = (hx(plsc.load_gather(sa, [adr_of(0, 0)]), 0),
                    hx(plsc.load_gather(sa, [adr_of(1, 0)]), 1),
                    hx(plsc.load_gather(sb, [adr_of(0, 0)]), 0),
                    hx(plsc.load_gather(sb, [adr_of(1, 0)]), 1))
            lax.fori_loop(0, vregs2, cb, init, unroll=2)

        def scan2():
            def sb(j, carry):
                ca, cb, ha, hb = carry
                sa = plsc.cumsum(ha)
                sb_ = plsc.cumsum(hb)
                nha = h_a[pl.ds((j + 1) * _LANES, _LANES)]
                nhb = h_b[pl.ds((j + 1) * _LANES, _LANES)]
                h_a[pl.ds(j * _LANES, _LANES)] = sa - ha + ca
                h_b[pl.ds(j * _LANES, _LANES)] = sb_ - hb + cb
                return (ca + sa[_LANES - 1], cb + sb_[_LANES - 1],
                        nha, nhb)
            lax.fori_loop(0, 2 * _RADIX, sb,
                          (jnp.int32(0), jnp.int32(0),
                           h_a[pl.ds(0, _LANES)],
                           h_b[pl.ds(0, _LANES)]),
                          unroll=2)

        def perm2(pref_a, pref_b, emit_a, emit_b):
            def pb(j, carry):
                (ka0, ha0), (ka1, ha1), (kb0, hb0), (kb1, hb1) = carry
                oa0 = plsc.load_gather(h_a, [ha0])
                na0 = pref_a(0, j + 1)
                oa1 = plsc.load_gather(h_a, [ha1])
                na1 = pref_a(1, j + 1)
                ob0 = plsc.load_gather(h_b, [hb0])
                nb0 = pref_b(0, j + 1)
                ob1 = plsc.load_gather(h_b, [hb1])
                nb1 = pref_b(1, j + 1)
                emit_a(0, j, ka0, oa0)
                plsc.store_scatter(h_a, [ha0], oa0 + 1)
                emit_a(1, j, ka1, oa1)
                plsc.store_scatter(h_a, [ha1], oa1 + 1)
                emit_b(0, j, kb0, ob0)
                plsc.store_scatter(h_b, [hb0], ob0 + 1)
                emit_b(1, j, kb1, ob1)
                plsc.store_scatter(h_b, [hb1], ob1 + 1)
                return (na0, na1, nb0, nb1)
            init = (pref_a(0, 0), pref_a(1, 0), pref_b(0, 0), pref_b(1, 0))
            lax.fori_loop(0, vregs2, pb, init, unroll=2)

        def move_pref(src, dg):
            def pref(h, j):
                k = plsc.load_gather(src, [adr_of(h, j)])
                return (k, dg(k) * (2 * _LANES) + lane + h * _LANES)
            return pref

        def move_emit(kdst, pay=None):
            def emit(h, j, k, ofs):
                fo = skew(ofs)
                plsc.store_scatter(kdst, [fo], k)
                if pay is not None:
                    plsc.store_scatter(pay, [fo], pos_of(h, j))
            return emit

        def pack_pref(ksrc, psrc):
            def pref(h, j):
                adr = adr_of(h, j)
                k = plsc.load_gather(ksrc, [adr])
                p = plsc.load_gather(psrc, [adr])
                pk = (((k >> 16) & 0xFFFF) << psh) | p
                return (pk, dig[1](k) * (2 * _LANES) + lane + h * _LANES)
            return pref

        def out_emit(outdst):
            def emit(h, j, pk, ofs):
                hi = ofs >= half_n
                m = jnp.where(hi, ofs - half_n, (half_n - 1) - ofs)
                z = plsc.load_gather(qv, [m])
                z = jnp.where(hi, z, -z)
                plsc.store_scatter(outdst, [pk & pmask],
                                   plsc.bitcast(z, jnp.int32))
            return emit

        for t in range(pairs):
            ca = wid * cpw + 2 * t
            cb = ca + 1
            pltpu.sync_copy(xt_hbm.at[ca], a_s.at[pl.ds(0, n)])
            pltpu.sync_copy(xt_hbm.at[cb], b_s.at[pl.ds(0, n)])
            relayout2(a_s, b_s)

            zero2()
            count2(a_s, b_s, dig[0])
            scan2()
            perm2(move_pref(a_s, dig[0]), move_pref(b_s, dig[0]),
                  move_emit(a_k, pay=a_p), move_emit(b_k, pay=b_p))

            zero2()
            count2(a_k, b_k, dig[1])
            scan2()
            perm2(pack_pref(a_k, a_p), pack_pref(b_k, b_p),
                  move_emit(a_s), move_emit(b_s))

            zero2()
            count2(a_s, b_s, dig[2])
            scan2()
            perm2(move_pref(a_s, dig[2]), move_pref(b_s, dig[2]),
                  move_emit(a_k), move_emit(b_k))

            zero2()
            count2(a_k, b_k, dig[3])
            scan2()
            perm2(move_pref(a_k, dig[3]), move_pref(b_k, dig[3]),
                  out_emit(a_p), out_emit(b_p))

            pltpu.sync_copy(a_p.at[pl.ds(0, n)], out_hbm.at[ca])
            pltpu.sync_copy(b_p.at[pl.ds(0, n)], out_hbm.at[cb])

    return body


def kernel(x):
    q = _icdf_table(_N)
    xt = lax.bitcast_convert_type(x, jnp.int32).T
    zt = _build_rank_kernel(_N, _D)(xt, q)
    return lax.bitcast_convert_type(zt.T, jnp.float32)


# speedup vs baseline: 1.1158x; 1.0389x over previous
import functools

import jax
import jax.numpy as jnp
from jax import lax
from jax.experimental import pallas as pl
from jax.experimental.pallas import tpu as pltpu
from jax.experimental.pallas import tpu_sc as plsc

_N = 16384
_D = 128
_NC = 2
_NS = 16
_LANES = 16
_RADIX = 256


_P0 = (-5.99633501014107895267E1, 9.80010754185999661536E1,
       -5.66762857469070293439E1, 1.39312609387279679503E1,
       -1.23916583867381258016E0)
_Q0 = (1.0, 1.95448858338141759834E0, 4.67627912898881538453E0,
       8.63602421390890590575E1, -2.25462687854119370527E2,
       2.00260212380060660359E2, -8.20372256168333339912E1,
       1.59056225126211695515E1, -1.18331621121330003142E0)
_P1 = (4.05544892305962419923E0, 3.15251094599893866154E1,
       5.71628192246421288162E1, 4.40805073893200834700E1,
       1.46849561928858024014E1, 2.18663306850790267539E0,
       -1.40256079171354495875E-1, -3.50424626827848203418E-2,
       -8.57456785154685413611E-4)
_Q1 = (1.0, 1.57799883256466749731E1, 4.53907635128879210584E1,
       4.13172038254672030440E1, 1.50425385692907503408E1,
       2.50464946208309415979E0, -1.42182922854787788574E-1,
       -3.80806407691578277194E-2, -9.33259480895457427372E-4)
_P2 = (3.23774891776946035970E0, 6.91522889068984211695E0,
       3.93881025292474443415E0, 1.33303460815807542389E0,
       2.01485389549179081538E-1, 1.23716634817820021358E-2,
       3.01581553508235416007E-4, 2.65806974686737550832E-6,
       6.23974539184983293730E-9)
_Q2 = (1.0, 6.02427039364742014255E0, 3.67983563856160859403E0,
       1.37702099489081330271E0, 2.16236993594496635890E-1,
       1.34204006088543189037E-2, 3.28014464682127739104E-4,
       2.89247864745380683936E-6, 6.79019408009981274425E-9)
_EXPM2 = 0.1353352832366127
_S2PI = 2.5066282746310002


def _polyval(coefs, x):
    r = jnp.full_like(x, coefs[0])
    for c in coefs[1:]:
        r = r * x + c
    return r


def _ndtri(p):
    mcp = jnp.where(p > 1.0 - _EXPM2, 1.0 - p, p)
    w = mcp - 0.5
    ww = w * w
    big = (w + w * ww * (_polyval(_P0, ww) / _polyval(_Q0, ww))) * (-_S2PI)
    z = jnp.sqrt(-2.0 * jnp.log(mcp))
    ft = z - jnp.log(z) / z
    zi = 1.0 / z
    small = ft - _polyval(_P2, zi) / _polyval(_Q2, zi) * zi
    other = ft - _polyval(_P1, zi) / _polyval(_Q1, zi) * zi
    x = jnp.where(mcp > _EXPM2, big, jnp.where(z >= 8.0, small, other))
    return jnp.where(p > 1.0 - _EXPM2, x, -x)


def _icdf_table(n):
    half = n // 2
    rows = half // 128

    def body(q_ref):
        r = lax.broadcasted_iota(jnp.int32, (rows, 128), 0)
        c = lax.broadcasted_iota(jnp.int32, (rows, 128), 1)
        k = (r * 128 + c + half).astype(jnp.float32)
        u = (k + 1.0) / float(n + 1)
        q_ref[...] = _ndtri(u)

    q = pl.pallas_call(
        body, out_shape=jax.ShapeDtypeStruct((rows, 128), jnp.float32))()
    return q.reshape(half)


def _build_rank_kernel(n, d, nc=_NC, ns=_NS, interpret=False):
    nw = nc * ns
    cpw = d // nw
    pairs = cpw // 2
    vregs = n // _LANES
    vregs2 = vregs // 2
    shv2 = vregs2.bit_length() - 1
    nsk = n + 2 * _LANES
    hh2 = _RADIX * _LANES
    pmask = n - 1
    psh = n.bit_length() - 1
    half_n = n // 2

    mesh = plsc.VectorSubcoreMesh(
        core_axis_name="c", subcore_axis_name="s",
        num_cores=nc, num_subcores=ns)

    @functools.partial(
        pl.kernel,
        out_type=jax.ShapeDtypeStruct((d, n), jnp.int32),
        mesh=mesh,
        interpret=interpret,
        compiler_params=pltpu.CompilerParams(needs_layout_passes=False),
        scratch_types=[
            pltpu.VMEM((nsk,), jnp.int32),
            pltpu.VMEM((nsk,), jnp.int32),
            pltpu.VMEM((nsk,), jnp.int32),
            pltpu.VMEM((nsk,), jnp.int32),
            pltpu.VMEM((nsk,), jnp.int32),
            pltpu.VMEM((nsk,), jnp.int32),
            pltpu.VMEM((hh2 + _LANES,), jnp.int32),
            pltpu.VMEM((hh2 + _LANES,), jnp.int32),
            pltpu.VMEM((half_n,), jnp.float32),
        ],
    )
    def body(xt_hbm, q_hbm, out_hbm, a_s, a_k, a_p, b_s, b_k, b_p,
             h_a, h_b, qv):
        wid = lax.axis_index("s") * nc + lax.axis_index("c")
        lane = lax.iota(jnp.int32, _LANES)
        lane_sp = lane * (vregs2 + 1)
        ch_off = _LANES * (vregs2 + 1)
        lane_s2 = lane * vregs2
        ones = jnp.ones((_LANES,), jnp.int32)

        pltpu.sync_copy(q_hbm, qv)

        def skew(pos):
            return pos + (pos >> shv2)

        def to_key(b):
            return b ^ ((b >> 31) & jnp.int32(0x7FFFFFFF))

        dig = (
            lambda k: k & 255,
            lambda k: (k >> 8) & 255,
            lambda k: (k >> psh) & 255,
            lambda k: ((k >> (psh + 8)) & 255) ^ 128,
        )

        def pos_of(h, j):
            return lane_s2 + h * half_n + j

        def adr_of(h, j):
            return lane_sp + h * ch_off + j

        def relayout2(sa, sb):
            def pre(j):
                return (to_key(sa[pl.ds(j * _LANES, _LANES)]),
                        to_key(sb[pl.ds(j * _LANES, _LANES)]))

            def rb(i, carry):
                ka, kb = carry
                j = vregs - 1 - i
                nxt = pre(j - 1)
                fo = skew(j * _LANES + lane)
                plsc.store_scatter(sa, [fo], ka)
                plsc.store_scatter(sb, [fo], kb)
                return nxt
            ka, kb = lax.fori_loop(0, vregs - 1, rb, pre(vregs - 1),
                                   unroll=2)
            plsc.store_scatter(sa, [lane], ka)
            plsc.store_scatter(sb, [lane], kb)

        def zero2():
            def zb(j, c):
                z16 = jnp.zeros((_LANES,), jnp.int32)
                h_a[pl.ds(j * _LANES, _LANES)] = z16
                h_b[pl.ds(j * _LANES, _LANES)] = z16
                return c
            lax.fori_loop(0, _RADIX, zb, 0, unroll=8)

        ones_hi = jnp.full((_LANES,), 65536, jnp.int32)

        def count2(sa, sb, dg):
            def hx(k):
                return dg(k) * _LANES + lane

            def cb(j, carry):
                ha0, ha1, hb0, hb1 = carry
                plsc.addupdate_scatter(h_a, [ha0], ones)
                na0 = plsc.load_gather(sa, [adr_of(0, j + 1)])
                plsc.addupdate_scatter(h_a, [ha1], ones_hi)
                na1 = plsc.load_gather(sa, [adr_of(1, j + 1)])
                plsc.addupdate_scatter(h_b, [hb0], ones)
                nb0 = plsc.load_gather(sb, [adr_of(0, j + 1)])
                plsc.addupdate_scatter(h_b, [hb1], ones_hi)
                nb1 = plsc.load_gather(sb, [adr_of(1, j + 1)])
                return (hx(na0), hx(na1), hx(nb0), hx(nb1))
            init = (hx(plsc.load_gather(sa, [adr_of(0, 0)])),
                    hx(plsc.load_gather(sa, [adr_of(1, 0)])),
                    hx(plsc.load_gather(sb, [adr_of(0, 0)])),
                    hx(plsc.load_gather(sb, [adr_of(1, 0)])))
            lax.fori_loop(0, vregs2, cb, init, unroll=2)

        def scan2():
            def one(w, carry):
                lo = w & 0xFFFF
                hi = lax.shift_right_logical(w, jnp.full((_LANES,), 16,
                                                         jnp.int32))
                slo = plsc.cumsum(lo)
                shi = plsc.cumsum(hi)
                tlo = slo[_LANES - 1]
                elo = slo - lo + carry
                ehi = shi - hi + (carry + tlo)
                out = elo | (ehi << 16)
                return out, carry + tlo + shi[_LANES - 1]

            def sb(j, carry):
                ca, cb, ha, hb = carry
                nha = h_a[pl.ds((j + 1) * _LANES, _LANES)]
                nhb = h_b[pl.ds((j + 1) * _LANES, _LANES)]
                oa, ca = one(ha, ca)
                ob, cb = one(hb, cb)
                h_a[pl.ds(j * _LANES, _LANES)] = oa
                h_b[pl.ds(j * _LANES, _LANES)] = ob
                return (ca, cb, nha, nhb)
            lax.fori_loop(0, _RADIX, sb,
                          (jnp.int32(0), jnp.int32(0),
                           h_a[pl.ds(0, _LANES)],
                           h_b[pl.ds(0, _LANES)]),
                          unroll=2)

        def perm2(pref_a, pref_b, emit_a, emit_b):
            c16 = jnp.full((_LANES,), 16, jnp.int32)

            def ext(w, h):
                return (w & 0xFFFF) if h == 0 else \
                    lax.shift_right_logical(w, c16)

            def pb(j, carry):
                (ka0, ha0), (ka1, ha1), (kb0, hb0), (kb1, hb1) = carry
                oa0 = ext(plsc.load_gather(h_a, [ha0]), 0)
                na0 = pref_a(0, j + 1)
                oa1 = ext(plsc.load_gather(h_a, [ha1]), 1)
                na1 = pref_a(1, j + 1)
                ob0 = ext(plsc.load_gather(h_b, [hb0]), 0)
                nb0 = pref_b(0, j + 1)
                ob1 = ext(plsc.load_gather(h_b, [hb1]), 1)
                nb1 = pref_b(1, j + 1)
                emit_a(0, j, ka0, oa0)
                plsc.addupdate_scatter(h_a, [ha0], ones)
                emit_a(1, j, ka1, oa1)
                plsc.addupdate_scatter(h_a, [ha1], ones_hi)
                emit_b(0, j, kb0, ob0)
                plsc.addupdate_scatter(h_b, [hb0], ones)
                emit_b(1, j, kb1, ob1)
                plsc.addupdate_scatter(h_b, [hb1], ones_hi)
                return (na0, na1, nb0, nb1)
            init = (pref_a(0, 0), pref_a(1, 0), pref_b(0, 0), pref_b(1, 0))
            lax.fori_loop(0, vregs2, pb, init, unroll=2)

        def move_pref(src, dg):
            def pref(h, j):
                k = plsc.load_gather(src, [adr_of(h, j)])
                return (k, dg(k) * _LANES + lane)
            return pref

        def move_emit(kdst, pay=None):
            def emit(h, j, k, ofs):
                fo = skew(ofs)
                plsc.store_scatter(kdst, [fo], k)
                if pay is not None:
                    plsc.store_scatter(pay, [fo], pos_of(h, j))
            return emit

        def pack_pref(ksrc, psrc):
            def pref(h, j):
                adr = adr_of(h, j)
                k = plsc.load_gather(ksrc, [adr])
                p = plsc.load_gather(psrc, [adr])
                pk = (((k >> 16) & 0xFFFF) << psh) | p
                return (pk, dig[1](k) * _LANES + lane)
            return pref

        def out_emit(outdst):
            def emit(h, j, pk, ofs):
                hi = ofs >= half_n
                m = jnp.where(hi, ofs - half_n, (half_n - 1) - ofs)
                z = plsc.load_gather(qv, [m])
                z = jnp.where(hi, z, -z)
                plsc.store_scatter(outdst, [pk & pmask],
                                   plsc.bitcast(z, jnp.int32))
            return emit

        for t in range(pairs):
            ca = wid * cpw + 2 * t
            cb = ca + 1
            pltpu.sync_copy(xt_hbm.at[ca], a_s.at[pl.ds(0, n)])
            pltpu.sync_copy(xt_hbm.at[cb], b_s.at[pl.ds(0, n)])
            relayout2(a_s, b_s)

            zero2()
            count2(a_s, b_s, dig[0])
            scan2()
            perm2(move_pref(a_s, dig[0]), move_pref(b_s, dig[0]),
                  move_emit(a_k, pay=a_p), move_emit(b_k, pay=b_p))

            zero2()
            count2(a_k, b_k, dig[1])
            scan2()
            perm2(pack_pref(a_k, a_p), pack_pref(b_k, b_p),
                  move_emit(a_s), move_emit(b_s))

            zero2()
            count2(a_s, b_s, dig[2])
            scan2()
            perm2(move_pref(a_s, dig[2]), move_pref(b_s, dig[2]),
                  move_emit(a_k), move_emit(b_k))

            zero2()
            count2(a_k, b_k, dig[3])
            scan2()
            perm2(move_pref(a_k, dig[3]), move_pref(b_k, dig[3]),
                  out_emit(a_p), out_emit(b_p))

            pltpu.sync_copy(a_p.at[pl.ds(0, n)], out_hbm.at[ca])
            pltpu.sync_copy(b_p.at[pl.ds(0, n)], out_hbm.at[cb])

    return body


def kernel(x):
    q = _icdf_table(_N)
    xt = lax.bitcast_convert_type(x, jnp.int32).T
    zt = _build_rank_kernel(_N, _D)(xt, q)
    return lax.bitcast_convert_type(zt.T, jnp.float32)
